# Initial kernel scaffold; baseline (speedup 1.0000x reference)
import jax
import jax.numpy as jnp
from jax.experimental import pallas as pl


def kernel(inp1, inp2):
    raise NotImplementedError("write your pallas kernel here")


# speedup vs baseline: 5.1358x; 5.1358x over previous
import jax
import jax.numpy as jnp
from jax.experimental import pallas as pl
from jax.experimental.pallas import tpu as pltpu

_N = 16384
_D = 128
_K = 64
_B = 2048


def _body(a_ref, s_ref, idx_ref, out_ref):
    a = a_ref[...]
    s = s_ref[...]
    idx = idx_ref[0]

    iota = jax.lax.broadcasted_iota(jnp.int32, (_K, _B), 0)
    idx_c = jnp.clip(idx, 0, _K - 1)
    onehot = (iota == idx_c).astype(jnp.float32)

    g = jax.lax.dot_general(
        s, onehot, (((1,), (0,)), ((), ())),
        preferred_element_type=jnp.float32)

    out_ref[...] = (a * g * g).T


def kernel(inp1, inp2):
    idx = inp2.astype(jnp.int32).reshape(_N // _B, 1, _B)
    table = jax.lax.slice(inp1, (0, 0), (_D, _K))
    out = pl.pallas_call(
        _body,
        grid=(_N // _B,),
        in_specs=[
            pl.BlockSpec((_D, _B), lambda i: (0, i)),
            pl.BlockSpec((_D, _K), lambda i: (0, 0)),
            pl.BlockSpec((1, 1, _B), lambda i: (i, 0, 0)),
        ],
        out_specs=pl.BlockSpec((_B, _D), lambda i: (i, 0)),
        out_shape=jax.ShapeDtypeStruct((_N, _D), jnp.float32),
        compiler_params=pltpu.CompilerParams(
            dimension_semantics=("parallel",)),
    )(inp1, table, idx)
    return (out,)
